# baseline (speedup 1.0000x reference)
import jax
import jax.numpy as jnp
from jax.experimental import pallas as pl
from jax.experimental.pallas import tpu as pltpu

_B, _N, _IN, _OUT, _H = 8, 1024, 128, 64, 4


def _fused_gat_kernel(x_ref, adj_ref, w1b_ref, ws_ref, wd_ref, wout_ref,
                      aout_ref, out_ref, adjf_ref):
    x = x_ref[0]
    adjf_ref[...] = adj_ref[0].astype(jnp.bfloat16)
    adjf = adjf_ref[...]

    xb = x.astype(jnp.bfloat16)
    whb = jnp.dot(xb, w1b_ref[...],
                  preferred_element_type=jnp.float32).astype(jnp.bfloat16)
    es = jnp.dot(x, ws_ref[...], preferred_element_type=jnp.float32)
    ed = jnp.dot(x, wd_ref[...], preferred_element_type=jnp.float32)
    ones = jnp.ones((_N, 1), jnp.bfloat16)

    est = es.T
    edt = ed.T
    u1 = jnp.exp(est).T.astype(jnp.bfloat16)
    u2 = jnp.exp(0.2 * est).T.astype(jnp.bfloat16)
    v1 = jnp.exp(edt).astype(jnp.bfloat16)
    v2 = jnp.exp(0.2 * edt).astype(jnp.bfloat16)

    acc = jnp.zeros((_N, _OUT), jnp.float32)
    for h in range(_H):
        t1 = u1[:, h:h + 1] * v1[h:h + 1, :]
        t2 = u2[:, h:h + 1] * v2[h:h + 1, :]
        p = jnp.maximum(t1, t2) * adjf
        whc = jnp.concatenate([whb[:, h * _OUT:(h + 1) * _OUT], ones], axis=1)
        hps = jnp.dot(p, whc, preferred_element_type=jnp.float32)
        hp = hps[:, :_OUT] * (1.0 / hps[:, _OUT:_OUT + 1])
        acc = acc + jnp.where(hp > 0, hp, jnp.exp(hp) - 1.0)

    x2 = acc * (1.0 / _H)
    wh2 = jnp.dot(x2, wout_ref[...], preferred_element_type=jnp.float32)
    e2 = jnp.dot(wh2, aout_ref[...], preferred_element_type=jnp.float32)
    u1o = jnp.exp(e2[:, 0:1]).astype(jnp.bfloat16)
    u2o = jnp.exp(0.2 * e2[:, 0:1]).astype(jnp.bfloat16)
    v1o = jnp.exp(e2[:, 1:2]).T.astype(jnp.bfloat16)
    v2o = jnp.exp(0.2 * e2[:, 1:2]).T.astype(jnp.bfloat16)
    t1 = u1o * v1o
    t2 = u2o * v2o
    p = jnp.maximum(t1, t2) * adjf
    whc2 = jnp.concatenate([wh2.astype(jnp.bfloat16), ones], axis=1)
    os = jnp.dot(p, whc2, preferred_element_type=jnp.float32)
    o = os[:, :_OUT] * (1.0 / os[:, _OUT:_OUT + 1])
    out_ref[0] = jnp.maximum(o, 0.0)


def kernel(input_feature, adj, W1, a1, W_out, a_out):
    w1r = jnp.transpose(W1, (1, 0, 2)).reshape(_IN, _H * _OUT)
    a_src = a1[:, :_OUT, 0]
    a_dst = a1[:, _OUT:, 0]
    eye = jnp.eye(_H, dtype=jnp.float32)
    a1s = (eye[:, None, :] * a_src[:, :, None]).reshape(_H * _OUT, _H)
    a1d = (eye[:, None, :] * a_dst[:, :, None]).reshape(_H * _OUT, _H)
    ws = w1r @ a1s
    wd = w1r @ a1d
    w1b = w1r.astype(jnp.bfloat16)
    aout2 = a_out.reshape(2, _OUT).T

    return pl.pallas_call(
        _fused_gat_kernel,
        grid=(_B,),
        in_specs=[
            pl.BlockSpec((1, _N, _IN), lambda b: (b, 0, 0)),
            pl.BlockSpec((1, _N, _N), lambda b: (b, 0, 0)),
            pl.BlockSpec((_IN, _H * _OUT), lambda b: (0, 0)),
            pl.BlockSpec((_IN, _H), lambda b: (0, 0)),
            pl.BlockSpec((_IN, _H), lambda b: (0, 0)),
            pl.BlockSpec((_OUT, _OUT), lambda b: (0, 0)),
            pl.BlockSpec((_OUT, 2), lambda b: (0, 0)),
        ],
        out_specs=pl.BlockSpec((1, _N, _OUT), lambda b: (b, 0, 0)),
        out_shape=jax.ShapeDtypeStruct((_B, _N, _OUT), jnp.float32),
        scratch_shapes=[
            pltpu.VMEM((_N, _N), jnp.bfloat16),
        ],
    )(input_feature, adj, w1b, ws, wd, W_out, aout2)
